# baseline (speedup 1.0000x reference)
import functools

import jax
import jax.numpy as jnp
from jax import lax
from jax.experimental import pallas as pl
from jax.experimental.pallas import tpu as pltpu
from jax.experimental.pallas import tpu_sc as plsc

NC = 2
NS = 16
L = 16
NW = NC * NS

N_ROWS = 8192
EMB = 1024

B_PER_W = N_ROWS // NW
CH = 16
NCHUNK = B_PER_W // CH
NBUF = 4

_mesh = plsc.VectorSubcoreMesh(core_axis_name="c", subcore_axis_name="s")


@functools.partial(
    pl.kernel,
    mesh=_mesh,
    out_type=jax.ShapeDtypeStruct((N_ROWS, EMB), jnp.float32),
    scratch_types=[
        pltpu.VMEM((L,), jnp.int32),
        pltpu.VMEM((B_PER_W,), jnp.int32),
        pltpu.VMEM((NBUF, CH, EMB), jnp.float32),
    ]
    + [pltpu.SemaphoreType.DMA] * (2 * NBUF),
)
def _sc_lookup(params_hbm, table_hbm, out_hbm, params_v, idx_v, rows_v,
               *sems):
    gsems, wsems = sems[:NBUF], sems[NBUF:]
    wid = lax.axis_index("s") * NC + lax.axis_index("c")
    base = wid * B_PER_W

    pltpu.sync_copy(params_hbm, params_v)
    pv = params_v[...]
    seq_len = pv[0]
    start_pos = pv[1]
    last = jnp.minimum(seq_len - 1, N_ROWS - 1)

    for g in range(B_PER_W // L):
        v = lax.iota(jnp.int32, L) + (base + g * L)
        v = jnp.minimum(v, last) + start_pos
        v = jnp.clip(v, 0, N_ROWS - 1)
        idx_v[pl.ds(g * L, L)] = v

    src0 = jnp.clip(jnp.minimum(base, last) + start_pos, 0, N_ROWS - 1)
    src_end = jnp.clip(
        jnp.minimum(base + B_PER_W - 1, last) + start_pos, 0, N_ROWS - 1)
    contiguous = jnp.logical_and(
        (src_end - src0) == (B_PER_W - 1), src0 % 8 == 0)

    def run_pipeline(start_gather, buf, ch, nchunk, nbuf):
        gathers = {}
        writes = {}
        for j in range(min(nbuf, nchunk)):
            gathers[j] = start_gather(j)
        for j in range(nchunk):
            slot = j % nbuf
            gathers[j].wait()
            writes[j] = pltpu.async_copy(
                buf(slot), out_hbm.at[pl.ds(base + j * ch, ch), :],
                wsems[slot])
            nj = j + nbuf
            if nj < nchunk:
                writes[nj - nbuf].wait()
                gathers[nj] = start_gather(nj)
        for j in range(max(0, nchunk - nbuf), nchunk):
            writes[j].wait()

    @pl.when(contiguous)
    def _fast():
        src0a = pl.multiple_of(src0, 8)

        def start_linear(j):
            return pltpu.async_copy(
                table_hbm.at[pl.ds(src0a + j * CH, CH), :],
                rows_v.at[j % NBUF], gsems[j % NBUF])

        run_pipeline(start_linear, lambda slot: rows_v.at[slot],
                     CH, NCHUNK, NBUF)

    @pl.when(jnp.logical_not(contiguous))
    def _general():
        def start_indirect(j):
            return pltpu.async_copy(
                table_hbm.at[idx_v.at[pl.ds(j * CH, CH)]],
                rows_v.at[j % NBUF], gsems[j % NBUF])

        run_pipeline(start_indirect, lambda slot: rows_v.at[slot],
                     CH, NCHUNK, NBUF)


def kernel(seq_len, start_pos, table):
    params = jnp.zeros((L,), jnp.int32)
    params = params.at[0].set(jnp.asarray(seq_len, jnp.int32))
    params = params.at[1].set(jnp.asarray(start_pos, jnp.int32))
    return _sc_lookup(params, table)


# speedup vs baseline: 1.0299x; 1.0299x over previous
import functools

import jax
import jax.numpy as jnp
from jax import lax
from jax.experimental import pallas as pl
from jax.experimental.pallas import tpu as pltpu
from jax.experimental.pallas import tpu_sc as plsc

NC = 2
NS = 16
L = 16
NW = NC * NS

N_ROWS = 8192
EMB = 1024

B_PER_W = N_ROWS // NW
CH = 16
NCHUNK = B_PER_W // CH
NBUF = 7

_mesh = plsc.VectorSubcoreMesh(core_axis_name="c", subcore_axis_name="s")


@functools.partial(
    pl.kernel,
    mesh=_mesh,
    out_type=jax.ShapeDtypeStruct((N_ROWS, EMB), jnp.float32),
    scratch_types=[
        pltpu.VMEM((L,), jnp.int32),
        pltpu.VMEM((B_PER_W,), jnp.int32),
        pltpu.VMEM((NBUF, CH, EMB), jnp.float32),
    ]
    + [pltpu.SemaphoreType.DMA] * (2 * NBUF),
)
def _sc_lookup(params_hbm, table_hbm, out_hbm, params_v, idx_v, rows_v,
               *sems):
    gsems, wsems = sems[:NBUF], sems[NBUF:]
    wid = lax.axis_index("s") * NC + lax.axis_index("c")
    base = wid * B_PER_W

    pltpu.sync_copy(params_hbm, params_v)
    pv = params_v[...]
    seq_len = pv[0]
    start_pos = pv[1]
    last = jnp.minimum(seq_len - 1, N_ROWS - 1)

    for g in range(B_PER_W // L):
        v = lax.iota(jnp.int32, L) + (base + g * L)
        v = jnp.minimum(v, last) + start_pos
        v = jnp.clip(v, 0, N_ROWS - 1)
        idx_v[pl.ds(g * L, L)] = v

    src0 = jnp.clip(jnp.minimum(base, last) + start_pos, 0, N_ROWS - 1)
    src_end = jnp.clip(
        jnp.minimum(base + B_PER_W - 1, last) + start_pos, 0, N_ROWS - 1)
    contiguous = jnp.logical_and(
        (src_end - src0) == (B_PER_W - 1), src0 % 8 == 0)

    def run_pipeline(start_gather, buf, ch, nchunk, nbuf):
        gathers = {}
        writes = {}
        for j in range(min(nbuf, nchunk)):
            gathers[j] = start_gather(j)
        for j in range(nchunk):
            slot = j % nbuf
            gathers[j].wait()
            writes[j] = pltpu.async_copy(
                buf(slot), out_hbm.at[pl.ds(base + j * ch, ch), :],
                wsems[slot])
            nj = j + nbuf
            if nj < nchunk:
                writes[nj - nbuf].wait()
                gathers[nj] = start_gather(nj)
        for j in range(max(0, nchunk - nbuf), nchunk):
            writes[j].wait()

    @pl.when(contiguous)
    def _fast():
        src0a = pl.multiple_of(src0, 8)

        def start_linear(j):
            return pltpu.async_copy(
                table_hbm.at[pl.ds(src0a + j * CH, CH), :],
                rows_v.at[j % NBUF], gsems[j % NBUF])

        run_pipeline(start_linear, lambda slot: rows_v.at[slot],
                     CH, NCHUNK, NBUF)

    @pl.when(jnp.logical_not(contiguous))
    def _general():
        def start_indirect(j):
            return pltpu.async_copy(
                table_hbm.at[idx_v.at[pl.ds(j * CH, CH)]],
                rows_v.at[j % NBUF], gsems[j % NBUF])

        run_pipeline(start_indirect, lambda slot: rows_v.at[slot],
                     CH, NCHUNK, NBUF)


def kernel(seq_len, start_pos, table):
    params = jnp.zeros((L,), jnp.int32)
    params = params.at[0].set(jnp.asarray(seq_len, jnp.int32))
    params = params.at[1].set(jnp.asarray(start_pos, jnp.int32))
    return _sc_lookup(params, table)
